# baseline (speedup 1.0000x reference)
import functools

import jax
import jax.numpy as jnp
from jax.experimental import pallas as pl
from jax.experimental.pallas import tpu as pltpu

F = 8


def _body(params_ref, hista_ref, histb_ref, w_ref, supp_ref, acc_ref, tot_ref, bmat_ref, *, T, PR, PH, PW):
    b = pl.program_id(0)
    t = pl.program_id(1)
    RT = PR * F
    W = PW * F

    @pl.when((b == 0) & (t == 0))
    def _init_b():
        rr = jax.lax.broadcasted_iota(jnp.int32, (W, PW), 0)
        cc = jax.lax.broadcasted_iota(jnp.int32, (W, PW), 1)
        bmat_ref[:] = (jnp.floor_divide(rr, F) == cc).astype(jnp.float32)

    ar = jax.lax.broadcasted_iota(jnp.int32, (PR, RT), 0)
    ac = jax.lax.broadcasted_iota(jnp.int32, (PR, RT), 1)
    amat = (jnp.floor_divide(ac, F) == ar).astype(jnp.float32)

    xa = hista_ref[0, 0]
    s1a = jnp.dot(amat, xa, preferred_element_type=jnp.float32)
    pa = jnp.dot(s1a, bmat_ref[:], preferred_element_type=jnp.float32)
    acc_ref[pl.ds(t * PR, PR), :] = pa

    xb = histb_ref[0, 0]
    s1b = jnp.dot(amat, xb, preferred_element_type=jnp.float32)
    pb = jnp.dot(s1b, bmat_ref[:], preferred_element_type=jnp.float32)
    acc_ref[pl.ds(PH // 2 + t * PR, PR), :] = pb

    tile_sum = jnp.sum(pa) + jnp.sum(pb)

    @pl.when(t == 0)
    def _init():
        tot_ref[0] = tile_sum

    @pl.when(t != 0)
    def _accum():
        tot_ref[0] = tot_ref[0] + tile_sum

    @pl.when(t == T - 1)
    def _finish():
        total = jnp.maximum(tot_ref[0], 1e-12)
        w_ref[0] = acc_ref[:] * (1.0 / total)
        px0 = params_ref[b, 0]
        sx = params_ref[b, 1]
        py0 = params_ref[b, 2]
        sy = params_ref[b, 3]
        NTile = PH * PW // 128
        CH = 128
        tt = jax.lax.broadcasted_iota(jnp.int32, (CH, 2, 128), 0)
        cd = jax.lax.broadcasted_iota(jnp.int32, (CH, 2, 128), 1)
        ln = jax.lax.broadcasted_iota(jnp.int32, (CH, 2, 128), 2)

        def _chunk(i, carry):
            g = (i * CH + tt) * 128 + ln
            xval = px0 + (jnp.remainder(g, PW).astype(jnp.float32) + 0.5) * sx
            yval = py0 + (jnp.floor_divide(g, PW).astype(jnp.float32) + 0.5) * sy
            supp_ref[0, pl.ds(i * CH, CH)] = jnp.where(cd == 0, xval, yval)
            return carry

        jax.lax.fori_loop(0, NTile // CH, _chunk, 0)


def kernel(histogram, x_lims, y_lims, shift):
    B, H, W = histogram.shape
    PH, PW = H // F, W // F
    RT = min(512, H // 2)
    T = (H // 2) // RT
    PR = RT // F

    sx = (x_lims[:, 1] - x_lims[:, 0]) / PW
    sy = (y_lims[:, 1] - y_lims[:, 0]) / PH
    px0 = x_lims[:, 0] + shift[0, 0, 0]
    py0 = y_lims[:, 0] + shift[0, 0, 1]
    params = jnp.stack([px0, sx, py0, sy], axis=-1)
    hist4 = histogram.reshape(B, 2, H // 2, W)

    w, supp4 = pl.pallas_call(
        functools.partial(_body, T=T, PR=PR, PH=PH, PW=PW),
        grid=(B, T),
        in_specs=[
            pl.BlockSpec((B, 4), lambda b, t: (0, 0), memory_space=pltpu.SMEM),
            pl.BlockSpec((1, 1, RT, W), lambda b, t: (b, 0, t, 0)),
            pl.BlockSpec((1, 1, RT, W), lambda b, t: (b, 1, t, 0)),
        ],
        out_specs=[
            pl.BlockSpec((1, PH, PW), lambda b, t: (b, 0, 0)),
            pl.BlockSpec((1, PH * PW // 128, 2, 128), lambda b, t: (b, 0, 0, 0)),
        ],
        out_shape=[
            jax.ShapeDtypeStruct((B, PH, PW), jnp.float32),
            jax.ShapeDtypeStruct((B, PH * PW // 128, 2, 128), jnp.float32),
        ],
        scratch_shapes=[
            pltpu.VMEM((PH, PW), jnp.float32),
            pltpu.SMEM((1,), jnp.float32),
            pltpu.VMEM((W, PW), jnp.float32),
        ],
        compiler_params=pltpu.CompilerParams(
            dimension_semantics=("arbitrary", "arbitrary"),
        ),
    )(params, hist4, hist4)

    supp = supp4.swapaxes(2, 3).reshape(B, PH * PW, 2)
    weights = w.reshape(B, PH * PW)
    return supp, weights


# speedup vs baseline: 1.0146x; 1.0146x over previous
import functools

import jax
import jax.numpy as jnp
from jax.experimental import pallas as pl
from jax.experimental.pallas import tpu as pltpu

F = 8


def _body(params_ref, hist_ref, w_ref, supp_ref, acc_ref, tot_ref, bmat_ref, *, T, PR, PH, PW):
    b = pl.program_id(0)
    t = pl.program_id(1)
    RT = PR * F
    W = PW * F

    @pl.when((b == 0) & (t == 0))
    def _init_b():
        rr = jax.lax.broadcasted_iota(jnp.int32, (W, PW), 0)
        cc = jax.lax.broadcasted_iota(jnp.int32, (W, PW), 1)
        bmat_ref[:] = (jnp.floor_divide(rr, F) == cc).astype(jnp.float32)

    x = hist_ref[0]
    ar = jax.lax.broadcasted_iota(jnp.int32, (PR, RT), 0)
    ac = jax.lax.broadcasted_iota(jnp.int32, (PR, RT), 1)
    amat = (jnp.floor_divide(ac, F) == ar).astype(jnp.float32)
    s1 = jnp.dot(amat, x, preferred_element_type=jnp.float32)
    pooled = jnp.dot(s1, bmat_ref[:], preferred_element_type=jnp.float32)
    acc_ref[pl.ds(t * PR, PR), :] = pooled
    tile_sum = jnp.sum(pooled)

    @pl.when(t == 0)
    def _init():
        tot_ref[0] = tile_sum

    @pl.when(t != 0)
    def _accum():
        tot_ref[0] = tot_ref[0] + tile_sum

    @pl.when(t == T - 1)
    def _finish():
        total = jnp.maximum(tot_ref[0], 1e-12)
        w_ref[0] = acc_ref[:] * (1.0 / total)
        px0 = params_ref[b, 0]
        sx = params_ref[b, 1]
        py0 = params_ref[b, 2]
        sy = params_ref[b, 3]
        NTile = PH * PW // 128
        CH = 128
        tt = jax.lax.broadcasted_iota(jnp.int32, (CH, 2, 128), 0)
        cd = jax.lax.broadcasted_iota(jnp.int32, (CH, 2, 128), 1)
        ln = jax.lax.broadcasted_iota(jnp.int32, (CH, 2, 128), 2)

        def _chunk(i, carry):
            g = (i * CH + tt) * 128 + ln
            xval = px0 + (jnp.remainder(g, PW).astype(jnp.float32) + 0.5) * sx
            yval = py0 + (jnp.floor_divide(g, PW).astype(jnp.float32) + 0.5) * sy
            supp_ref[0, pl.ds(i * CH, CH)] = jnp.where(cd == 0, xval, yval)
            return carry

        jax.lax.fori_loop(0, NTile // CH, _chunk, 0)


def kernel(histogram, x_lims, y_lims, shift):
    B, H, W = histogram.shape
    PH, PW = H // F, W // F
    RT = min(1024, H)
    T = H // RT
    PR = RT // F

    sx = (x_lims[:, 1] - x_lims[:, 0]) / PW
    sy = (y_lims[:, 1] - y_lims[:, 0]) / PH
    px0 = x_lims[:, 0] + shift[0, 0, 0]
    py0 = y_lims[:, 0] + shift[0, 0, 1]
    params = jnp.stack([px0, sx, py0, sy], axis=-1)

    w, supp4 = pl.pallas_call(
        functools.partial(_body, T=T, PR=PR, PH=PH, PW=PW),
        grid=(B, T),
        in_specs=[
            pl.BlockSpec((B, 4), lambda b, t: (0, 0), memory_space=pltpu.SMEM),
            pl.BlockSpec((1, RT, W), lambda b, t: (b, t, 0)),
        ],
        out_specs=[
            pl.BlockSpec((1, PH, PW), lambda b, t: (b, 0, 0)),
            pl.BlockSpec((1, PH * PW // 128, 2, 128), lambda b, t: (b, 0, 0, 0)),
        ],
        out_shape=[
            jax.ShapeDtypeStruct((B, PH, PW), jnp.float32),
            jax.ShapeDtypeStruct((B, PH * PW // 128, 2, 128), jnp.float32),
        ],
        scratch_shapes=[
            pltpu.VMEM((PH, PW), jnp.float32),
            pltpu.SMEM((1,), jnp.float32),
            pltpu.VMEM((W, PW), jnp.float32),
        ],
        compiler_params=pltpu.CompilerParams(
            dimension_semantics=("arbitrary", "arbitrary"),
        ),
    )(params, histogram)

    supp = supp4.swapaxes(2, 3).reshape(B, PH * PW, 2)
    weights = w.reshape(B, PH * PW)
    return supp, weights


# speedup vs baseline: 1.3313x; 1.3121x over previous
import functools

import jax
import jax.numpy as jnp
from jax.experimental import pallas as pl
from jax.experimental.pallas import tpu as pltpu

F = 8


def _body(params_ref, hist_ref, w_ref, supp_ref, acc_ref, tot_ref, bmat_ref, *, T, PR, PH, PW):
    b = pl.program_id(0)
    t = pl.program_id(1)
    RT = PR * F
    W = PW * F

    @pl.when((b == 0) & (t == 0))
    def _init_b():
        rr = jax.lax.broadcasted_iota(jnp.int32, (W, PW), 0)
        cc = jax.lax.broadcasted_iota(jnp.int32, (W, PW), 1)
        bmat_ref[:] = (jnp.floor_divide(rr, F) == cc).astype(jnp.float32)

    x = hist_ref[0]
    ar = jax.lax.broadcasted_iota(jnp.int32, (PR, RT), 0)
    ac = jax.lax.broadcasted_iota(jnp.int32, (PR, RT), 1)
    amat = (jnp.floor_divide(ac, F) == ar).astype(jnp.float32)
    s1 = jnp.dot(amat, x, preferred_element_type=jnp.float32)
    pooled = jnp.dot(s1, bmat_ref[:], preferred_element_type=jnp.float32)
    acc_ref[pl.ds(t * PR, PR), :] = pooled
    tile_sum = jnp.sum(pooled)

    @pl.when(t == 0)
    def _init():
        tot_ref[0] = tile_sum

    @pl.when(t != 0)
    def _accum():
        tot_ref[0] = tot_ref[0] + tile_sum

    px0 = params_ref[b, 0]
    sx = params_ref[b, 1]
    py0 = params_ref[b, 2]
    sy = params_ref[b, 3]
    NU = PH * PW // 512
    CH = NU // T
    uu = jax.lax.broadcasted_iota(jnp.int32, (CH, 8, 128), 0)
    ss = jax.lax.broadcasted_iota(jnp.int32, (CH, 8, 128), 1)
    ll = jax.lax.broadcasted_iota(jnp.int32, (CH, 8, 128), 2)
    g = ((t * CH + uu) * 4 + jnp.floor_divide(ss, 2)) * 128 + ll
    xval = px0 + (jnp.remainder(g, PW).astype(jnp.float32) + 0.5) * sx
    yval = py0 + (jnp.floor_divide(g, PW).astype(jnp.float32) + 0.5) * sy
    supp_ref[0, pl.ds(t * CH, CH)] = jnp.where(ss % 2 == 0, xval, yval)

    @pl.when(t == T - 1)
    def _finish():
        total = jnp.maximum(tot_ref[0], 1e-12)
        w_ref[0] = acc_ref[:] * (1.0 / total)


def kernel(histogram, x_lims, y_lims, shift):
    B, H, W = histogram.shape
    PH, PW = H // F, W // F
    RT = min(1024, H)
    T = H // RT
    PR = RT // F

    sx = (x_lims[:, 1] - x_lims[:, 0]) / PW
    sy = (y_lims[:, 1] - y_lims[:, 0]) / PH
    px0 = x_lims[:, 0] + shift[0, 0, 0]
    py0 = y_lims[:, 0] + shift[0, 0, 1]
    params = jnp.stack([px0, sx, py0, sy], axis=-1)

    w, supp4 = pl.pallas_call(
        functools.partial(_body, T=T, PR=PR, PH=PH, PW=PW),
        grid=(B, T),
        in_specs=[
            pl.BlockSpec((B, 4), lambda b, t: (0, 0), memory_space=pltpu.SMEM),
            pl.BlockSpec((1, RT, W), lambda b, t: (b, t, 0)),
        ],
        out_specs=[
            pl.BlockSpec((1, PH, PW), lambda b, t: (b, 0, 0)),
            pl.BlockSpec((1, PH * PW // 512, 8, 128), lambda b, t: (b, 0, 0, 0)),
        ],
        out_shape=[
            jax.ShapeDtypeStruct((B, PH, PW), jnp.float32),
            jax.ShapeDtypeStruct((B, PH * PW // 512, 8, 128), jnp.float32),
        ],
        scratch_shapes=[
            pltpu.VMEM((PH, PW), jnp.float32),
            pltpu.SMEM((1,), jnp.float32),
            pltpu.VMEM((W, PW), jnp.float32),
        ],
        compiler_params=pltpu.CompilerParams(
            dimension_semantics=("arbitrary", "arbitrary"),
        ),
    )(params, histogram)

    supp = supp4.reshape(B, PH * PW // 128, 2, 128).swapaxes(2, 3).reshape(B, PH * PW, 2)
    weights = w.reshape(B, PH * PW)
    return supp, weights
